# baseline (speedup 1.0000x reference)
import functools

import jax
import jax.numpy as jnp
from jax import lax
from jax.experimental import pallas as pl
from jax.experimental.pallas import tpu as pltpu
from jax.experimental.pallas import tpu_sc as plsc

NC = 2
NS = 16
NW = NC * NS

BATCH = 524288
DIM = 32
FLAT = BATCH * DIM
PER_W = FLAT // NW
CHUNK = 32768
NCHUNK = PER_W // CHUNK
UNROLL = 8
STEPS = CHUNK // (16 * UNROLL)

_mesh = plsc.VectorSubcoreMesh(core_axis_name="c", subcore_axis_name="s")


@functools.partial(
    pl.kernel,
    mesh=_mesh,
    out_type=jax.ShapeDtypeStruct((NW * DIM,), jnp.float32),
    scratch_types=[
        pltpu.VMEM((CHUNK,), jnp.float32),
        pltpu.VMEM((CHUNK,), jnp.float32),
        pltpu.VMEM((DIM,), jnp.float32),
        pltpu.SemaphoreType.DMA,
        pltpu.SemaphoreType.DMA,
    ],
)
def _colsum_sc(x_hbm, out_hbm, buf0, buf1, stage, sem0, sem1):
    wid = lax.axis_index("s") * NC + lax.axis_index("c")
    base = wid * PER_W
    bufs = (buf0, buf1)
    sems = (sem0, sem1)
    copies = [
        pltpu.async_copy(x_hbm.at[pl.ds(base + c * CHUNK, CHUNK)], bufs[c], sems[c])
        for c in range(2)
    ]
    zero = jnp.zeros((16,), jnp.float32)
    accs = (zero,) * UNROLL
    for c in range(NCHUNK):
        b = c % 2
        copies[b].wait()
        buf = bufs[b]

        def body(t, a, buf=buf):
            off = t * (16 * UNROLL)
            return tuple(
                a[k] + buf[pl.ds(off + 16 * k, 16)] for k in range(UNROLL)
            )

        accs = lax.fori_loop(0, STEPS, body, accs)
        nxt = c + 2
        if nxt < NCHUNK:
            copies[b] = pltpu.async_copy(
                x_hbm.at[pl.ds(base + nxt * CHUNK, CHUNK)], bufs[b], sems[b]
            )
    acc_even = (accs[0] + accs[2]) + (accs[4] + accs[6])
    acc_odd = (accs[1] + accs[3]) + (accs[5] + accs[7])
    stage[pl.ds(0, 16)] = acc_even
    stage[pl.ds(16, 16)] = acc_odd
    pltpu.sync_copy(stage, out_hbm.at[pl.ds(wid * DIM, DIM)])


def _codebook_tc(p_ref, y_ref, o_ref):
    p = p_ref[...]
    s = jnp.sum(p, axis=0, keepdims=True)
    y = y_ref[...]
    m = jnp.sum(y * s, axis=1, keepdims=True)
    q = jnp.sum(y * y, axis=1, keepdims=True)
    metric = jnp.sign(m) * (m * m) / q
    maxv = jnp.max(metric)
    row = lax.broadcasted_iota(jnp.int32, metric.shape, 0)
    cand = jnp.where(metric == maxv, row, 2**30)
    o_ref[0, 0] = jnp.min(cand)


def kernel(inputs, mean_distances):
    flat = inputs.reshape(FLAT)
    partials = _colsum_sc(flat)
    idx = pl.pallas_call(
        _codebook_tc,
        out_shape=jax.ShapeDtypeStruct((1, 1), jnp.int32),
        out_specs=pl.BlockSpec(memory_space=pltpu.SMEM),
    )(partials.reshape(NW, DIM), mean_distances)
    return idx.reshape(1)
